# baseline (speedup 1.0000x reference)
import functools
import math

import jax
import jax.numpy as jnp
import numpy as np
from jax import lax
from jax.experimental import pallas as pl
from jax.experimental.pallas import tpu as pltpu
from jax.experimental.pallas import tpu_sc as plsc

AVG_NUM_NEIGHBORS = 32.0

NC = 2
NS = 16
NW = NC * NS
CH = 128


def _tc_node_prep(node_attrs, node_feats, W_up, Wsk_t, R10):
    N, D = node_feats.shape
    A = node_attrs.shape[1]
    L = R10.shape[1]
    BLK = 2000 if N % 2000 == 0 else N

    def body(na_ref, nf_ref, wup_ref, wsk_ref, r_ref, q_ref, sc_ref):
        nf = nf_ref[...]
        h = jnp.dot(nf, wup_ref[...], preferred_element_type=jnp.float32)
        aR = jnp.dot(na_ref[...], r_ref[...], preferred_element_type=jnp.float32)
        hu = lax.bitcast_convert_type(h.astype(jnp.bfloat16), jnp.uint16).astype(jnp.uint32)
        au = lax.bitcast_convert_type(aR.astype(jnp.bfloat16), jnp.uint16).astype(jnp.uint32)
        q_ref[...] = lax.bitcast_convert_type((hu << 16) | au, jnp.int32)
        acc = jnp.zeros((BLK, D), jnp.float32)
        for a in range(A):
            na_a = na_ref[:, a : a + 1]
            acc = acc + jnp.dot(nf * na_a, wsk_ref[a], preferred_element_type=jnp.float32)
        sc_ref[...] = acc

    return pl.pallas_call(
        body,
        grid=(N // BLK,),
        in_specs=[
            pl.BlockSpec((BLK, A), lambda i: (i, 0)),
            pl.BlockSpec((BLK, D), lambda i: (i, 0)),
            pl.BlockSpec((D, D), lambda i: (0, 0)),
            pl.BlockSpec((A, D, D), lambda i: (0, 0, 0)),
            pl.BlockSpec((A, L), lambda i: (0, 0)),
        ],
        out_specs=[
            pl.BlockSpec((BLK, D), lambda i: (i, 0)),
            pl.BlockSpec((BLK, D), lambda i: (i, 0)),
        ],
        out_shape=[
            jax.ShapeDtypeStruct((N, D), jnp.int32),
            jax.ShapeDtypeStruct((N, D), jnp.float32),
        ],
    )(node_attrs, node_feats, W_up, Wsk_t, R10)


NBUF = 6


def _worker_span(wid, nchunks):
    cw = nchunks // NW
    rem = nchunks % NW
    n = cw + jnp.where(wid < rem, 1, 0)
    base_chunk = wid * cw + jnp.minimum(wid, rem)
    return base_chunk * CH, n


def _sc_gather(edge_index, Q, Eg, base_e):
    W = Q.shape[1]
    nchunks = Eg // CH
    mesh = plsc.VectorSubcoreMesh(core_axis_name="c", subcore_axis_name="s")

    @functools.partial(
        pl.kernel,
        out_type=jax.ShapeDtypeStruct((Eg, W), jnp.int32),
        mesh=mesh,
        scratch_types=[
            pltpu.VMEM((NBUF, CH), jnp.int32),
            pltpu.VMEM((NBUF, CH, W), jnp.int32),
            pltpu.SemaphoreType.DMA((NBUF,)),
            pltpu.SemaphoreType.DMA((NBUF,)),
            pltpu.SemaphoreType.DMA((NBUF,)),
        ],
    )
    def run(ei_hbm, q_hbm, qs_out, idx_v, qrow_v, isem, gsem, osem):
        wid = lax.axis_index("s") * NC + lax.axis_index("c")
        base, n = _worker_span(wid, nchunks)

        def idx_copy(j, s):
            return pltpu.make_async_copy(
                ei_hbm.at[0, pl.ds(base_e + base + j * CH, CH)], idx_v.at[s], isem.at[s]
            )

        def gather_copy(j, s):
            return pltpu.make_async_copy(q_hbm.at[idx_v.at[s]], qrow_v.at[s], gsem.at[s])

        def out_copy(j, s):
            return pltpu.make_async_copy(
                qrow_v.at[s], qs_out.at[pl.ds(base + j * CH, CH)], osem.at[s]
            )

        for k in range(NBUF - 1):
            idx_copy(k, k).start()
        for k in range(NBUF - 2):
            idx_copy(k, k).wait()
            gather_copy(k, k).start()

        def body(j, carry):
            s = j % NBUF
            gather_copy(j, s).wait()
            out_copy(j, s).start()

            @pl.when(j >= 2)
            def _():
                out_copy(j - 2, (j - 2) % NBUF).wait()

            @pl.when(j + NBUF - 1 < n)
            def _():
                idx_copy(j + NBUF - 1, (j - 1) % NBUF).start()

            @pl.when(j + NBUF - 2 < n)
            def _():
                idx_copy(j + NBUF - 2, (j - 2) % NBUF).wait()
                gather_copy(j + NBUF - 2, (j - 2) % NBUF).start()

            return carry

        lax.fori_loop(0, n, body, 0)
        out_copy(n - 2, (n - 2) % NBUF).wait()
        out_copy(n - 1, (n - 1) % NBUF).wait()

    return run(edge_index, Q)


EBLK = 2560


def _tc_pack_edges(ef, ea, F, lo_blk, nblk):
    G = EBLK // 8

    def body(ef_ref, ea_ref, y_ref):
        parts = []
        zeros = jnp.zeros((G, 16 - F - 1), jnp.float32)
        for t in range(8):
            parts.append(ef_ref[t * G : (t + 1) * G, :])
            parts.append(ea_ref[t * G : (t + 1) * G, :])
            parts.append(zeros)
        y_ref[...] = jnp.concatenate(parts, axis=1)

    return pl.pallas_call(
        body,
        grid=(nblk,),
        in_specs=[
            pl.BlockSpec((EBLK, F), lambda i: (i + lo_blk, 0)),
            pl.BlockSpec((EBLK, 1), lambda i: (i + lo_blk, 0)),
        ],
        out_specs=pl.BlockSpec((G, 128), lambda i: (i, 0)),
        out_shape=jax.ShapeDtypeStruct((nblk * G, 128), jnp.float32),
    )(ef, ea)


def _tc_edge_compute(qs, Y, T, W128, F):
    Ep, W = qs.shape
    L = W128.shape[0]
    D = W128.shape[1]
    G = EBLK // 8
    assert Ep % EBLK == 0

    def body(qs_ref, y_ref, t_ref, w_ref, out_ref):
        y = y_ref[...]
        for t in range(8):
            yt = y[:, 16 * t : 16 * (t + 1)]
            eft = yt[:, 0:F]
            eat = yt[:, F : F + 1]
            qu = lax.bitcast_convert_type(qs_ref[t * G : (t + 1) * G, :], jnp.uint32)
            hs = lax.bitcast_convert_type((qu >> 16).astype(jnp.uint16), jnp.bfloat16)
            aR = lax.bitcast_convert_type(qu.astype(jnp.uint16), jnp.bfloat16)
            v = aR.astype(jnp.float32) * jnp.dot(
                eft, t_ref[...], preferred_element_type=jnp.float32
            )
            tp = jnp.dot(v, w_ref[...], preferred_element_type=jnp.float32)
            out_ref[t * G : (t + 1) * G, :] = tp * hs.astype(jnp.float32) * eat

    return pl.pallas_call(
        body,
        grid=(Ep // EBLK,),
        in_specs=[
            pl.BlockSpec((EBLK, W), lambda i: (i, 0)),
            pl.BlockSpec((G, 128), lambda i: (i, 0)),
            pl.BlockSpec((F, L), lambda i: (0, 0)),
            pl.BlockSpec((L, D), lambda i: (0, 0)),
        ],
        out_specs=pl.BlockSpec((EBLK, D), lambda i: (i, 0)),
        out_shape=jax.ShapeDtypeStruct((Ep, D), jnp.float32),
    )(qs, Y, T, W128)


SBUF = 2


def _sc_scatter(edge_index, mji, zeros_init, N, D, NP, base_e):
    Eg = mji.shape[0]
    nchunks = Eg // CH
    mesh = plsc.VectorSubcoreMesh(core_axis_name="c", subcore_axis_name="s")

    @functools.partial(
        pl.kernel,
        out_type=jax.ShapeDtypeStruct((NC, NP, D), jnp.float32),
        mesh=mesh,
        scratch_types=[
            pltpu.VMEM((SBUF, CH), jnp.int32),
            pltpu.VMEM((SBUF, CH, D), jnp.float32),
            pltpu.VMEM_SHARED((NP, D), jnp.float32),
            pltpu.SemaphoreType.DMA((SBUF,)),
            pltpu.SemaphoreType.DMA((SBUF,)),
        ],
    )
    def run(ei_hbm, mji_hbm, zeros_hbm, out_hbm, idx_v, row_v, acc, isem, msem):
        cid = lax.axis_index("c")
        sid = lax.axis_index("s")
        wid = sid * NC + cid
        rpt = NP // NS
        base, n = _worker_span(wid, nchunks)

        def idx_copy(j, s):
            return pltpu.make_async_copy(
                ei_hbm.at[1, pl.ds(base_e + base + j * CH, CH)], idx_v.at[s], isem.at[s]
            )

        def mji_copy(j, s):
            return pltpu.make_async_copy(
                mji_hbm.at[pl.ds(base + j * CH, CH)], row_v.at[s], msem.at[s]
            )

        for k in range(SBUF):
            idx_copy(k, k).start()
            mji_copy(k, k).start()
        pltpu.sync_copy(zeros_hbm, acc.at[pl.ds(sid * rpt, rpt)])
        plsc.subcore_barrier()

        def body(j, carry):
            s = j % SBUF
            idx_copy(j, s).wait()
            mji_copy(j, s).wait()
            pltpu.sync_copy(row_v.at[s], acc.at[idx_v.at[s]], add=True)

            @pl.when(j + SBUF < n)
            def _():
                idx_copy(j + SBUF, s).start()
                mji_copy(j + SBUF, s).start()

            return carry

        lax.fori_loop(0, n, body, 0)
        plsc.subcore_barrier()
        pltpu.sync_copy(acc.at[pl.ds(sid * rpt, rpt)], out_hbm.at[cid, pl.ds(sid * rpt, rpt)])

    return run(edge_index, mji, zeros_init)


def _tc_final(partials_a, partials_b, sc, W_lin_scaled):
    N, D = sc.shape
    BLK = 2000 if N % 2000 == 0 else N

    def body(pa_ref, pb_ref, sc_ref, wl_ref, out_ref):
        m = pa_ref[0] + pa_ref[1] + pb_ref[0] + pb_ref[1]
        out_ref[...] = jnp.dot(m, wl_ref[...], preferred_element_type=jnp.float32) + sc_ref[...]

    pspec = pl.BlockSpec((NC, BLK, D), lambda i: (0, i, 0))
    return pl.pallas_call(
        body,
        grid=(N // BLK,),
        in_specs=[
            pspec,
            pspec,
            pl.BlockSpec((BLK, D), lambda i: (i, 0)),
            pl.BlockSpec((D, D), lambda i: (0, 0)),
        ],
        out_specs=pl.BlockSpec((BLK, D), lambda i: (i, 0)),
        out_shape=jax.ShapeDtypeStruct((N, D), jnp.float32),
    )(partials_a, partials_b, sc, W_lin_scaled)


def kernel(node_attrs, node_feats, edge_attrs, edge_feats, edge_index, W_up, W_tpw, W_lin, W_skip):
    N, A = node_attrs.shape
    D = node_feats.shape[1]
    E, F = edge_feats.shape
    AP = 16

    assert E % CH == 0 and E % EBLK == 0 and E // CH >= NW * (NBUF + 1)

    Wsk_t = W_skip.transpose(1, 0, 2) / np.sqrt(float(D * A))
    W_lin_scaled = W_lin / AVG_NUM_NEIGHBORS
    L = AP * F
    R_np = np.zeros((A, L), np.float32)
    T_np = np.zeros((F, L), np.float32)
    for e in range(F):
        for a in range(A):
            R_np[a, a * F + e] = 1.0
        for a in range(AP):
            T_np[e, a * F + e] = 1.0
    R10 = jnp.asarray(R_np)
    T = jnp.asarray(T_np)
    W128 = jnp.concatenate(
        [W_tpw.reshape(A * F, D), jnp.zeros(((AP - A) * F, D), jnp.float32)], axis=0
    )

    Q, sc = _tc_node_prep(node_attrs, node_feats, W_up, Wsk_t, R10)

    nblk = E // EBLK
    E1 = (nblk // 2) * EBLK
    NP = -(-N // 128) * 128
    zeros_init = jnp.zeros((NP // NS, D), jnp.float32)

    def run_group(lo, hi):
        Yg = _tc_pack_edges(edge_feats, edge_attrs, F, lo // EBLK, (hi - lo) // EBLK)
        qs = _sc_gather(edge_index, Q, hi - lo, lo)
        mji = _tc_edge_compute(qs, Yg, T, W128, F)
        return _sc_scatter(edge_index, mji, zeros_init, N, D, NP, lo)

    partials_a = run_group(0, E1)
    partials_b = run_group(E1, E)
    return _tc_final(partials_a, partials_b, sc, W_lin_scaled)


# speedup vs baseline: 1.7866x; 1.7866x over previous
import functools
import math

import jax
import jax.numpy as jnp
import numpy as np
from jax import lax
from jax.experimental import pallas as pl
from jax.experimental.pallas import tpu as pltpu
from jax.experimental.pallas import tpu_sc as plsc

AVG_NUM_NEIGHBORS = 32.0

NC = 2
NS = 16
NW = NC * NS
CH = 128


def _tc_node_prep(node_attrs, node_feats, W_up, Wsk_t, R10):
    N, D = node_feats.shape
    A = node_attrs.shape[1]
    L = R10.shape[1]
    BLK = 2000 if N % 2000 == 0 else N

    def body(na_ref, nf_ref, wup_ref, wsk_ref, r_ref, q_ref, sc_ref):
        nf = nf_ref[...]
        h = jnp.dot(nf, wup_ref[...], preferred_element_type=jnp.float32)
        aR = jnp.dot(na_ref[...], r_ref[...], preferred_element_type=jnp.float32)
        hu = lax.bitcast_convert_type(h.astype(jnp.bfloat16), jnp.uint16).astype(jnp.uint32)
        au = lax.bitcast_convert_type(aR.astype(jnp.bfloat16), jnp.uint16).astype(jnp.uint32)
        q_ref[...] = lax.bitcast_convert_type((hu << 16) | au, jnp.int32)
        acc = jnp.zeros((BLK, D), jnp.float32)
        for a in range(A):
            na_a = na_ref[:, a : a + 1]
            acc = acc + jnp.dot(nf * na_a, wsk_ref[a], preferred_element_type=jnp.float32)
        sc_ref[...] = acc

    return pl.pallas_call(
        body,
        grid=(N // BLK,),
        in_specs=[
            pl.BlockSpec((BLK, A), lambda i: (i, 0)),
            pl.BlockSpec((BLK, D), lambda i: (i, 0)),
            pl.BlockSpec((D, D), lambda i: (0, 0)),
            pl.BlockSpec((A, D, D), lambda i: (0, 0, 0)),
            pl.BlockSpec((A, L), lambda i: (0, 0)),
        ],
        out_specs=[
            pl.BlockSpec((BLK, D), lambda i: (i, 0)),
            pl.BlockSpec((BLK, D), lambda i: (i, 0)),
        ],
        out_shape=[
            jax.ShapeDtypeStruct((N, D), jnp.int32),
            jax.ShapeDtypeStruct((N, D), jnp.float32),
        ],
    )(node_attrs, node_feats, W_up, Wsk_t, R10)


NBUF = 6


def _worker_span(wid, nchunks):
    cw = nchunks // NW
    rem = nchunks % NW
    n = cw + jnp.where(wid < rem, 1, 0)
    base_chunk = wid * cw + jnp.minimum(wid, rem)
    return base_chunk * CH, n


def _sc_gather(edge_index, Q, Eg, base_e):
    W = Q.shape[1]
    nchunks = Eg // CH
    mesh = plsc.VectorSubcoreMesh(core_axis_name="c", subcore_axis_name="s")

    @functools.partial(
        pl.kernel,
        out_type=jax.ShapeDtypeStruct((Eg, W), jnp.int32),
        mesh=mesh,
        scratch_types=[
            pltpu.VMEM((NBUF, CH), jnp.int32),
            pltpu.VMEM((NBUF, CH, W), jnp.int32),
            pltpu.SemaphoreType.DMA((NBUF,)),
            pltpu.SemaphoreType.DMA((NBUF,)),
            pltpu.SemaphoreType.DMA((NBUF,)),
        ],
    )
    def run(ei_hbm, q_hbm, qs_out, idx_v, qrow_v, isem, gsem, osem):
        wid = lax.axis_index("s") * NC + lax.axis_index("c")
        base, n = _worker_span(wid, nchunks)

        def idx_copy(j, s):
            return pltpu.make_async_copy(
                ei_hbm.at[0, pl.ds(base_e + base + j * CH, CH)], idx_v.at[s], isem.at[s]
            )

        def gather_copy(j, s):
            return pltpu.make_async_copy(q_hbm.at[idx_v.at[s]], qrow_v.at[s], gsem.at[s])

        def out_copy(j, s):
            return pltpu.make_async_copy(
                qrow_v.at[s], qs_out.at[pl.ds(base + j * CH, CH)], osem.at[s]
            )

        for k in range(NBUF - 1):
            idx_copy(k, k).start()
        for k in range(NBUF - 2):
            idx_copy(k, k).wait()
            gather_copy(k, k).start()

        def body(j, carry):
            s = j % NBUF
            gather_copy(j, s).wait()
            out_copy(j, s).start()

            @pl.when(j >= 2)
            def _():
                out_copy(j - 2, (j - 2) % NBUF).wait()

            @pl.when(j + NBUF - 1 < n)
            def _():
                idx_copy(j + NBUF - 1, (j - 1) % NBUF).start()

            @pl.when(j + NBUF - 2 < n)
            def _():
                idx_copy(j + NBUF - 2, (j - 2) % NBUF).wait()
                gather_copy(j + NBUF - 2, (j - 2) % NBUF).start()

            return carry

        lax.fori_loop(0, n, body, 0)
        out_copy(n - 2, (n - 2) % NBUF).wait()
        out_copy(n - 1, (n - 1) % NBUF).wait()

    return run(edge_index, Q)


EBLK = 2560


def _tc_edge_compute(qs, efs_T, T, W128, lo_blk):
    Ep, W = qs.shape
    F = efs_T.shape[0]
    L = W128.shape[0]
    D = W128.shape[1]
    assert Ep % EBLK == 0

    def body(qs_ref, efs_ref, t_ref, w_ref, out_ref):
        qu = lax.bitcast_convert_type(qs_ref[...], jnp.uint32)
        hs = lax.bitcast_convert_type((qu >> 16).astype(jnp.uint16), jnp.bfloat16)
        aR = lax.bitcast_convert_type(qu.astype(jnp.uint16), jnp.bfloat16)
        efT = lax.dot_general(
            efs_ref[...], t_ref[...], (((0,), (0,)), ((), ())),
            preferred_element_type=jnp.float32,
        )
        v = aR.astype(jnp.float32) * efT
        tp = jnp.dot(v, w_ref[...], preferred_element_type=jnp.float32)
        out_ref[...] = tp * hs.astype(jnp.float32)

    return pl.pallas_call(
        body,
        grid=(Ep // EBLK,),
        in_specs=[
            pl.BlockSpec((EBLK, W), lambda i: (i, 0)),
            pl.BlockSpec((F, EBLK), lambda i: (0, i + lo_blk)),
            pl.BlockSpec((F, L), lambda i: (0, 0)),
            pl.BlockSpec((L, D), lambda i: (0, 0)),
        ],
        out_specs=pl.BlockSpec((EBLK, D), lambda i: (i, 0)),
        out_shape=jax.ShapeDtypeStruct((Ep, D), jnp.float32),
    )(qs, efs_T, T, W128)


SBUF = 2


def _sc_scatter(edge_index, mji, zeros_init, N, D, NP, base_e):
    Eg = mji.shape[0]
    nchunks = Eg // CH
    mesh = plsc.VectorSubcoreMesh(core_axis_name="c", subcore_axis_name="s")

    @functools.partial(
        pl.kernel,
        out_type=jax.ShapeDtypeStruct((NC, NP, D), jnp.float32),
        mesh=mesh,
        scratch_types=[
            pltpu.VMEM((SBUF, CH), jnp.int32),
            pltpu.VMEM((SBUF, CH, D), jnp.float32),
            pltpu.VMEM_SHARED((NP, D), jnp.float32),
            pltpu.SemaphoreType.DMA((SBUF,)),
            pltpu.SemaphoreType.DMA((SBUF,)),
        ],
    )
    def run(ei_hbm, mji_hbm, zeros_hbm, out_hbm, idx_v, row_v, acc, isem, msem):
        cid = lax.axis_index("c")
        sid = lax.axis_index("s")
        wid = sid * NC + cid
        rpt = NP // NS
        base, n = _worker_span(wid, nchunks)

        def idx_copy(j, s):
            return pltpu.make_async_copy(
                ei_hbm.at[1, pl.ds(base_e + base + j * CH, CH)], idx_v.at[s], isem.at[s]
            )

        def mji_copy(j, s):
            return pltpu.make_async_copy(
                mji_hbm.at[pl.ds(base + j * CH, CH)], row_v.at[s], msem.at[s]
            )

        for k in range(SBUF):
            idx_copy(k, k).start()
            mji_copy(k, k).start()
        pltpu.sync_copy(zeros_hbm, acc.at[pl.ds(sid * rpt, rpt)])
        plsc.subcore_barrier()

        def body(j, carry):
            s = j % SBUF
            idx_copy(j, s).wait()
            mji_copy(j, s).wait()
            pltpu.sync_copy(row_v.at[s], acc.at[idx_v.at[s]], add=True)

            @pl.when(j + SBUF < n)
            def _():
                idx_copy(j + SBUF, s).start()
                mji_copy(j + SBUF, s).start()

            return carry

        lax.fori_loop(0, n, body, 0)
        plsc.subcore_barrier()
        pltpu.sync_copy(acc.at[pl.ds(sid * rpt, rpt)], out_hbm.at[cid, pl.ds(sid * rpt, rpt)])

    return run(edge_index, mji, zeros_init)


def _tc_final(partials_a, partials_b, sc, W_lin_scaled):
    N, D = sc.shape
    BLK = 2000 if N % 2000 == 0 else N

    def body(pa_ref, pb_ref, sc_ref, wl_ref, out_ref):
        m = pa_ref[0] + pa_ref[1] + pb_ref[0] + pb_ref[1]
        out_ref[...] = jnp.dot(m, wl_ref[...], preferred_element_type=jnp.float32) + sc_ref[...]

    pspec = pl.BlockSpec((NC, BLK, D), lambda i: (0, i, 0))
    return pl.pallas_call(
        body,
        grid=(N // BLK,),
        in_specs=[
            pspec,
            pspec,
            pl.BlockSpec((BLK, D), lambda i: (i, 0)),
            pl.BlockSpec((D, D), lambda i: (0, 0)),
        ],
        out_specs=pl.BlockSpec((BLK, D), lambda i: (i, 0)),
        out_shape=jax.ShapeDtypeStruct((N, D), jnp.float32),
    )(partials_a, partials_b, sc, W_lin_scaled)


def kernel(node_attrs, node_feats, edge_attrs, edge_feats, edge_index, W_up, W_tpw, W_lin, W_skip):
    N, A = node_attrs.shape
    D = node_feats.shape[1]
    E, F = edge_feats.shape
    AP = 16

    assert E % CH == 0 and E % EBLK == 0 and E // CH >= NW * (NBUF + 1)

    Wsk_t = W_skip.transpose(1, 0, 2) / np.sqrt(float(D * A))
    W_lin_scaled = W_lin / AVG_NUM_NEIGHBORS
    L = AP * F
    R_np = np.zeros((A, L), np.float32)
    T_np = np.zeros((F, L), np.float32)
    for e in range(F):
        for a in range(A):
            R_np[a, a * F + e] = 1.0
        for a in range(AP):
            T_np[e, a * F + e] = 1.0
    R10 = jnp.asarray(R_np)
    T = jnp.asarray(T_np)
    W128 = jnp.concatenate(
        [W_tpw.reshape(A * F, D), jnp.zeros(((AP - A) * F, D), jnp.float32)], axis=0
    )

    Q, sc = _tc_node_prep(node_attrs, node_feats, W_up, Wsk_t, R10)

    nblk = E // EBLK
    E1 = (nblk // 2) * EBLK
    NP = -(-N // 128) * 128
    zeros_init = jnp.zeros((NP // NS, D), jnp.float32)

    efs_T = (edge_feats * edge_attrs).T

    def run_group(lo, hi):
        qs = _sc_gather(edge_index, Q, hi - lo, lo)
        mji = _tc_edge_compute(qs, efs_T, T, W128, lo // EBLK)
        return _sc_scatter(edge_index, mji, zeros_init, N, D, NP, lo)

    partials_a = run_group(0, E1)
    partials_b = run_group(E1, E)
    return _tc_final(partials_a, partials_b, sc, W_lin_scaled)
